# baseline (speedup 1.0000x reference)
import functools

import jax
import jax.numpy as jnp
from jax import lax
from jax.experimental import pallas as pl
from jax.experimental.pallas import tpu as pltpu
from jax.experimental.pallas import tpu_sc as plsc

D_MODEL = 512
D_PE = 64
HEADS = 4
ROWS = 64 * 143
NC, NS = 2, 16
NW = NC * NS
PAD_ROWS = 9216
PER_W = (2 * PAD_ROWS) // NW
R_BLK = 1144
BATCH = 8


def _sc_gather(table, idx_all):
    mesh = plsc.VectorSubcoreMesh(core_axis_name="c", subcore_axis_name="s")

    @functools.partial(
        pl.kernel,
        mesh=mesh,
        out_type=jax.ShapeDtypeStruct((2 * PAD_ROWS, D_PE), jnp.float32),
        scratch_types=[
            pltpu.VMEM((PER_W,), jnp.int32),
            pltpu.VMEM((PER_W, D_PE), jnp.float32),
            pltpu.SemaphoreType.DMA,
        ],
        compiler_params=pltpu.CompilerParams(use_tc_tiling_on_sc=False),
    )
    def gather_kernel(table_hbm, idx_hbm, out_hbm, idx_v, rows_v, sem):
        wid = lax.axis_index("s") * NC + lax.axis_index("c")
        base = wid * PER_W
        pltpu.sync_copy(idx_hbm.at[pl.ds(base, PER_W)], idx_v)
        pltpu.async_copy(table_hbm.at[idx_v], rows_v, sem).wait()
        pltpu.sync_copy(rows_v, out_hbm.at[pl.ds(base, PER_W)])

    return gather_kernel(table, idx_all)


def _add_kernel(pf_ref, ps_ref, x_ref, o_ref):
    pe = jnp.concatenate([pf_ref[0], ps_ref[0]], axis=-1)
    o_ref[...] = x_ref[...] + jnp.tile(pe, (1, HEADS))[None]


def _tc_add(pe2, x3):
    grid = (ROWS // R_BLK, BATCH)
    return pl.pallas_call(
        _add_kernel,
        grid=grid,
        in_specs=[
            pl.BlockSpec((1, R_BLK, D_PE), lambda r, b: (0, r, 0)),
            pl.BlockSpec((1, R_BLK, D_PE), lambda r, b: (1, r, 0)),
            pl.BlockSpec((1, R_BLK, D_MODEL), lambda r, b: (b, r, 0)),
        ],
        out_specs=pl.BlockSpec((1, R_BLK, D_MODEL), lambda r, b: (b, r, 0)),
        out_shape=jax.ShapeDtypeStruct((BATCH, ROWS, D_MODEL), jnp.float32),
    )(pe2, pe2, x3)


def kernel(x, first_half_pe, second_half_pe, pjpe):
    B, T, J, D = x.shape
    idx1 = first_half_pe.reshape(-1).astype(jnp.int32)
    idx2 = second_half_pe.reshape(-1).astype(jnp.int32)
    pad = PAD_ROWS - ROWS
    idx_all = jnp.concatenate(
        [jnp.pad(idx1, (0, pad)), jnp.pad(idx2, (0, pad))])
    pe = _sc_gather(pjpe, idx_all)
    pe2 = pe.reshape(2, PAD_ROWS, D_PE)
    x3 = x.reshape(B, T * J, D)
    out = _tc_add(pe2, x3)
    return out.reshape(B, T, J, D)


# speedup vs baseline: 1.0012x; 1.0012x over previous
import functools

import jax
import jax.numpy as jnp
from jax import lax
from jax.experimental import pallas as pl
from jax.experimental.pallas import tpu as pltpu
from jax.experimental.pallas import tpu_sc as plsc

D_MODEL = 512
D_PE = 64
HEADS = 4
ROWS = 64 * 143
NC, NS = 2, 16
NW = NC * NS
PER_W = ROWS // NW
WIN = 296
R_BLK = 1144
BATCH = 8


def _sc_gather(table, idx1, idx2):
    mesh = plsc.VectorSubcoreMesh(core_axis_name="c", subcore_axis_name="s")

    @functools.partial(
        pl.kernel,
        mesh=mesh,
        out_type=(
            jax.ShapeDtypeStruct((ROWS, D_PE), jnp.float32),
            jax.ShapeDtypeStruct((ROWS, D_PE), jnp.float32),
        ),
        scratch_types=[
            pltpu.VMEM((WIN,), jnp.int32),
            pltpu.VMEM((WIN, D_PE), jnp.float32),
            pltpu.SemaphoreType.DMA,
        ],
        compiler_params=pltpu.CompilerParams(use_tc_tiling_on_sc=False),
    )
    def gather_kernel(table_hbm, idx1_hbm, idx2_hbm, out1_hbm, out2_hbm,
                      idx_v, rows_v, sem):
        wid = lax.axis_index("s") * NC + lax.axis_index("c")
        base = wid * PER_W
        win = jnp.minimum((base // 8) * 8, ROWS - WIN)
        win = pl.multiple_of(win, 8)
        d = base - win
        for idx_hbm, out_hbm in ((idx1_hbm, out1_hbm), (idx2_hbm, out2_hbm)):
            pltpu.sync_copy(idx_hbm.at[pl.ds(win, WIN)], idx_v)
            pltpu.async_copy(table_hbm.at[idx_v], rows_v, sem).wait()
            pltpu.sync_copy(rows_v.at[pl.ds(d, PER_W)],
                            out_hbm.at[pl.ds(base, PER_W)])

    return gather_kernel(table, idx1, idx2)


def _add_kernel(pf_ref, ps_ref, x_ref, o_ref):
    pe = jnp.concatenate([pf_ref[...], ps_ref[...]], axis=-1)
    o_ref[...] = x_ref[...] + jnp.tile(pe, (1, HEADS))[None]


def _tc_add(pf, ps, x3):
    grid = (ROWS // R_BLK, BATCH)
    return pl.pallas_call(
        _add_kernel,
        grid=grid,
        in_specs=[
            pl.BlockSpec((R_BLK, D_PE), lambda r, b: (r, 0)),
            pl.BlockSpec((R_BLK, D_PE), lambda r, b: (r, 0)),
            pl.BlockSpec((1, R_BLK, D_MODEL), lambda r, b: (b, r, 0)),
        ],
        out_specs=pl.BlockSpec((1, R_BLK, D_MODEL), lambda r, b: (b, r, 0)),
        out_shape=jax.ShapeDtypeStruct((BATCH, ROWS, D_MODEL), jnp.float32),
    )(pf, ps, x3)


def kernel(x, first_half_pe, second_half_pe, pjpe):
    B, T, J, D = x.shape
    idx1 = first_half_pe.reshape(-1).astype(jnp.int32)
    idx2 = second_half_pe.reshape(-1).astype(jnp.int32)
    pf, ps = _sc_gather(pjpe, idx1, idx2)
    x3 = x.reshape(B, T * J, D)
    out = _tc_add(pf, ps, x3)
    return out.reshape(B, T, J, D)


# speedup vs baseline: 1.0170x; 1.0157x over previous
import functools

import jax
import jax.numpy as jnp
from jax import lax
from jax.experimental import pallas as pl
from jax.experimental.pallas import tpu as pltpu
from jax.experimental.pallas import tpu_sc as plsc

D_MODEL = 512
D_PE = 64
HEADS = 4
ROWS = 64 * 143
NC, NS = 2, 16
NW = NC * NS
PER_W = ROWS // NW
WIN = 296
R_BLK = 1144
BATCH = 8


def _sc_gather(table, idx1, idx2):
    mesh = plsc.VectorSubcoreMesh(core_axis_name="c", subcore_axis_name="s")

    @functools.partial(
        pl.kernel,
        mesh=mesh,
        out_type=jax.ShapeDtypeStruct((ROWS, 2 * D_PE), jnp.float32),
        scratch_types=[
            pltpu.VMEM((WIN,), jnp.int32),
            pltpu.VMEM((WIN, D_PE), jnp.float32),
            pltpu.SemaphoreType.DMA,
        ],
        compiler_params=pltpu.CompilerParams(use_tc_tiling_on_sc=False),
    )
    def gather_kernel(table_hbm, idx1_hbm, idx2_hbm, out_hbm,
                      idx_v, rows_v, sem):
        wid = lax.axis_index("s") * NC + lax.axis_index("c")
        base = wid * PER_W
        win = jnp.minimum((base // 8) * 8, ROWS - WIN)
        win = pl.multiple_of(win, 8)
        d = base - win
        for half, idx_hbm in ((0, idx1_hbm), (1, idx2_hbm)):
            pltpu.sync_copy(idx_hbm.at[pl.ds(win, WIN)], idx_v)
            pltpu.async_copy(table_hbm.at[idx_v], rows_v, sem).wait()
            pltpu.sync_copy(
                rows_v.at[pl.ds(d, PER_W)],
                out_hbm.at[pl.ds(base, PER_W),
                           pl.ds(half * D_PE, D_PE)])

    return gather_kernel(table, idx1, idx2)


def _add_kernel(pe_ref, x_ref, o_ref):
    o_ref[...] = x_ref[...] + jnp.tile(pe_ref[...], (1, HEADS))[None]


def _tc_add(pe, x3):
    grid = (ROWS // R_BLK, BATCH)
    return pl.pallas_call(
        _add_kernel,
        grid=grid,
        in_specs=[
            pl.BlockSpec((R_BLK, 2 * D_PE), lambda r, b: (r, 0)),
            pl.BlockSpec((1, R_BLK, D_MODEL), lambda r, b: (b, r, 0)),
        ],
        out_specs=pl.BlockSpec((1, R_BLK, D_MODEL), lambda r, b: (b, r, 0)),
        out_shape=jax.ShapeDtypeStruct((BATCH, ROWS, D_MODEL), jnp.float32),
    )(pe, x3)


def kernel(x, first_half_pe, second_half_pe, pjpe):
    B, T, J, D = x.shape
    idx1 = first_half_pe.reshape(-1).astype(jnp.int32)
    idx2 = second_half_pe.reshape(-1).astype(jnp.int32)
    pe = _sc_gather(pjpe, idx1, idx2)
    x3 = x.reshape(B, T * J, D)
    out = _tc_add(pe, x3)
    return out.reshape(B, T, J, D)
